# baseline (speedup 1.0000x reference)
import jax
import jax.numpy as jnp
from jax import lax
from jax.experimental import pallas as pl
from jax.experimental.pallas import tpu as pltpu
from jax.experimental.pallas import tpu_sc as plsc

C = 65536
B = 4096
D = 64
R = 1024
K = C // R
PR = C // 128
IR = PR // K
SR = B // 128
PADR = 16
SROWS = SR + 2 * PADR
NT = 32
RPT = C // NT
CHUNK = 256
NCH = RPT // CHUNK


def _im_i0(i):
    z = jnp.int32(0)
    return (lax.convert_element_type(i, jnp.int32), z)


def _im_00(i):
    z = jnp.int32(0)
    return (z, z)


def _sc_body(q_hbm, s_hbm, buf0, buf1, si0, si1, so0, so1):
    cid = lax.axis_index("c")
    sid = lax.axis_index("s")
    wid = sid * 2 + cid
    base = wid * RPT
    bufs = (buf0, buf1)
    sin = (si0, si1)
    sout = (so0, so1)
    h_out = [None, None]
    for k in range(NCH):
        b = k % 2
        if h_out[b] is not None:
            h_out[b].wait()
        r0 = base + k * CHUNK
        pltpu.async_copy(q_hbm.at[pl.ds(r0, CHUNK), :], bufs[b], sin[b]).wait()
        h_out[b] = pltpu.async_copy(bufs[b], s_hbm.at[pl.ds(r0, CHUNK), :],
                                    sout[b])
    for h in h_out:
        h.wait()


def _sc_sampled(queue_embeddings):
    fn = pl.kernel(
        _sc_body,
        out_type=[jax.ShapeDtypeStruct((C, D), jnp.float32)],
        mesh=plsc.VectorSubcoreMesh(
            core_axis_name="c", subcore_axis_name="s",
            num_cores=2, num_subcores=16),
        scratch_types=[
            pltpu.VMEM((CHUNK, D), jnp.float32),
            pltpu.VMEM((CHUNK, D), jnp.float32),
            pltpu.SemaphoreType.DMA,
            pltpu.SemaphoreType.DMA,
            pltpu.SemaphoreType.DMA,
            pltpu.SemaphoreType.DMA,
        ],
    )
    (se,) = fn(queue_embeddings)
    return se


def _tc_body(p_ref, qe_ref, qlo_ref, qhi_ref, epad_ref, slo_ref, shi_ref,
             ne_ref, slo_out, shi_out, nlo_out, nhi_out):
    i = pl.program_id(0)
    p = p_ref[0]

    qe = qe_ref[...]
    d = i * R - p
    s0 = jnp.where(d < 0, d + C, d)
    a1 = R + jnp.minimum(s0, B)
    a2 = jnp.maximum(R + s0 - C, 0)
    e1 = epad_ref[pl.ds(a1, R), :]
    e2 = epad_ref[pl.ds(a2, R), :]
    r = lax.broadcasted_iota(jnp.int32, (R, 1), 0)
    pos = s0 + r
    wrap = pos >= C
    posm = jnp.where(wrap, pos - C, pos)
    mask = posm < B
    val = jnp.where(wrap, e2, e1)
    ne_ref[...] = jnp.where(mask, val, qe)

    qlo = qlo_ref[...]
    qhi = qhi_ref[...]
    slo_out[...] = qlo
    shi_out[...] = qhi
    q = p // 128
    lam = p - q * 128
    rowg = lax.broadcasted_iota(jnp.int32, (IR, 128), 0) + i * IR
    lane = lax.broadcasted_iota(jnp.int32, (IR, 128), 1)
    g = rowg * 128 + lane
    j = g - p
    wrp = j < 0
    jm = jnp.where(wrp, j + C, j)
    mask_i = jm < B
    start_a = jnp.clip(PADR + i * IR - q - 1, 0, SROWS - 16)
    start_w = jnp.clip(PADR + i * IR - q + PR - 1, 0, SROWS - 16)
    hi_lane = lane >= lam

    def pick(src_ref):
        s_a = pltpu.roll(src_ref[pl.ds(start_a, 16), :], lam, axis=1)
        s_w = pltpu.roll(src_ref[pl.ds(start_w, 16), :], lam, axis=1)
        v_a = jnp.where(hi_lane, s_a[1:1 + IR], s_a[0:IR])
        v_w = jnp.where(hi_lane, s_w[1:1 + IR], s_w[0:IR])
        return jnp.where(wrp, v_w, v_a)

    nlo_out[...] = jnp.where(mask_i, pick(slo_ref), qlo)
    nhi_out[...] = jnp.where(mask_i, pick(shi_ref), qhi)


def _split_planes(x64, rows):
    u = lax.bitcast_convert_type(x64, jnp.uint64)
    lo = lax.convert_element_type(u & jnp.uint64(0xFFFFFFFF), jnp.uint32)
    hi = lax.convert_element_type(u >> jnp.uint64(32), jnp.uint32)
    lo = lax.bitcast_convert_type(lo, jnp.int32).reshape(rows, 128)
    hi = lax.bitcast_convert_type(hi, jnp.int32).reshape(rows, 128)
    return lo, hi


def _join_planes(lo2d, hi2d):
    lo = lax.bitcast_convert_type(lo2d.reshape(-1), jnp.uint32)
    hi = lax.bitcast_convert_type(hi2d.reshape(-1), jnp.uint32)
    u = (lax.convert_element_type(hi, jnp.uint64) << jnp.uint64(32)) | \
        lax.convert_element_type(lo, jnp.uint64)
    return lax.bitcast_convert_type(u, jnp.int64)


def _pad_rows(x2d, pad):
    z = jnp.zeros((pad, 128), jnp.int32)
    return jnp.concatenate([z, x2d, z])


def _tc_new(p32, queue_embeddings, embeddings, queue_item_ids, item_ids):
    epad = jnp.concatenate([
        jnp.zeros((R, D), jnp.float32),
        embeddings,
        jnp.zeros((R, D), jnp.float32)])
    qlo, qhi = _split_planes(queue_item_ids, PR)
    slo, shi = _split_planes(item_ids, SR)
    slo, shi = _pad_rows(slo, PADR), _pad_rows(shi, PADR)

    ids2d = jax.ShapeDtypeStruct((PR, 128), jnp.int32)
    ne, s_lo, s_hi, n_lo, n_hi = pl.pallas_call(
        _tc_body,
        grid=(K,),
        in_specs=[
            pl.BlockSpec((1,), lambda i: (jnp.int32(0),),
                         memory_space=pltpu.SMEM),
            pl.BlockSpec((R, D), _im_i0),
            pl.BlockSpec((IR, 128), _im_i0),
            pl.BlockSpec((IR, 128), _im_i0),
            pl.BlockSpec((B + 2 * R, D), _im_00),
            pl.BlockSpec((SROWS, 128), _im_00),
            pl.BlockSpec((SROWS, 128), _im_00),
        ],
        out_specs=[
            pl.BlockSpec((R, D), _im_i0),
            pl.BlockSpec((IR, 128), _im_i0),
            pl.BlockSpec((IR, 128), _im_i0),
            pl.BlockSpec((IR, 128), _im_i0),
            pl.BlockSpec((IR, 128), _im_i0),
        ],
        out_shape=[
            jax.ShapeDtypeStruct((C, D), jnp.float32),
            ids2d, ids2d, ids2d, ids2d,
        ],
        compiler_params=pltpu.CompilerParams(dimension_semantics=("arbitrary",)),
    )(p32, queue_embeddings, qlo, qhi, epad, slo, shi)

    si = _join_planes(s_lo, s_hi)
    ni = _join_planes(n_lo, n_hi)
    return ne, si, ni


def kernel(embeddings, item_ids, queue_embeddings, queue_item_ids, ptr):
    p32 = jnp.mod(ptr, C).astype(jnp.int32).reshape((1,))
    se = _sc_sampled(queue_embeddings)
    ne, si, ni = _tc_new(p32, queue_embeddings, embeddings,
                         queue_item_ids, item_ids)
    return (se, si, ne, ni)


# speedup vs baseline: 1.0840x; 1.0840x over previous
import jax
import jax.numpy as jnp
from jax import lax
from jax.experimental import pallas as pl
from jax.experimental.pallas import tpu as pltpu
from jax.experimental.pallas import tpu_sc as plsc

C = 65536
B = 4096
D = 64
PR = C // 128
KI = 64
IR = PR // KI
SR = B // 128
PADR = 16
SROWS = SR + 2 * PADR
NT = 32
RPT = C // NT
CHUNK = 256
NCH = RPT // CHUNK
S = 512
KF = C // S
NB = B // S + 1


def _im_i0(i):
    z = jnp.int32(0)
    return (lax.convert_element_type(i, jnp.int32), z)


def _im_00(i):
    z = jnp.int32(0)
    return (z, z)


def _sc_body(q_hbm, s_hbm, n_hbm, buf0, buf1, si0, si1, so0, so1, to0, to1):
    cid = lax.axis_index("c")
    sid = lax.axis_index("s")
    wid = sid * 2 + cid
    base = wid * RPT
    bufs = (buf0, buf1)
    sin = (si0, si1)
    sout = (so0, so1)
    tout = (to0, to1)
    h_s = [None, None]
    h_n = [None, None]
    for k in range(NCH):
        b = k % 2
        if h_s[b] is not None:
            h_s[b].wait()
            h_n[b].wait()
        r0 = base + k * CHUNK
        pltpu.async_copy(q_hbm.at[pl.ds(r0, CHUNK), :], bufs[b], sin[b]).wait()
        h_s[b] = pltpu.async_copy(bufs[b], s_hbm.at[pl.ds(r0, CHUNK), :],
                                  sout[b])
        h_n[b] = pltpu.async_copy(bufs[b], n_hbm.at[pl.ds(r0, CHUNK), :],
                                  tout[b])
    for b in range(2):
        h_s[b].wait()
        h_n[b].wait()


def _sc_copies(queue_embeddings):
    emb2d = jax.ShapeDtypeStruct((C, D), jnp.float32)
    fn = pl.kernel(
        _sc_body,
        out_type=[emb2d, emb2d],
        mesh=plsc.VectorSubcoreMesh(
            core_axis_name="c", subcore_axis_name="s",
            num_cores=2, num_subcores=16),
        scratch_types=[
            pltpu.VMEM((CHUNK, D), jnp.float32),
            pltpu.VMEM((CHUNK, D), jnp.float32),
            pltpu.SemaphoreType.DMA,
            pltpu.SemaphoreType.DMA,
            pltpu.SemaphoreType.DMA,
            pltpu.SemaphoreType.DMA,
            pltpu.SemaphoreType.DMA,
            pltpu.SemaphoreType.DMA,
        ],
    )
    se, nb = fn(queue_embeddings)
    return se, nb


def _ids_body(p_ref, qlo_ref, qhi_ref, slo_ref, shi_ref,
              slo_out, shi_out, nlo_out, nhi_out):
    i = pl.program_id(0)
    p = p_ref[0]
    qlo = qlo_ref[...]
    qhi = qhi_ref[...]
    slo_out[...] = qlo
    shi_out[...] = qhi
    q = p // 128
    lam = p - q * 128
    rowg = lax.broadcasted_iota(jnp.int32, (IR, 128), 0) + i * IR
    lane = lax.broadcasted_iota(jnp.int32, (IR, 128), 1)
    g = rowg * 128 + lane
    j = g - p
    wrp = j < 0
    jm = jnp.where(wrp, j + C, j)
    mask_i = jm < B
    start_a = jnp.clip(PADR + i * IR - q - 1, 0, SROWS - 16)
    start_w = jnp.clip(PADR + i * IR - q + PR - 1, 0, SROWS - 16)
    hi_lane = lane >= lam

    def pick(src_ref):
        s_a = pltpu.roll(src_ref[pl.ds(start_a, 16), :], lam, axis=1)
        s_w = pltpu.roll(src_ref[pl.ds(start_w, 16), :], lam, axis=1)
        v_a = jnp.where(hi_lane, s_a[1:1 + IR], s_a[0:IR])
        v_w = jnp.where(hi_lane, s_w[1:1 + IR], s_w[0:IR])
        return jnp.where(wrp, v_w, v_a)

    nlo_out[...] = jnp.where(mask_i, pick(slo_ref), qlo)
    nhi_out[...] = jnp.where(mask_i, pick(shi_ref), qhi)


def _split_planes(x64, rows):
    u = lax.bitcast_convert_type(x64, jnp.uint64)
    lo = lax.convert_element_type(u & jnp.uint64(0xFFFFFFFF), jnp.uint32)
    hi = lax.convert_element_type(u >> jnp.uint64(32), jnp.uint32)
    lo = lax.bitcast_convert_type(lo, jnp.int32).reshape(rows, 128)
    hi = lax.bitcast_convert_type(hi, jnp.int32).reshape(rows, 128)
    return lo, hi


def _join_planes(lo2d, hi2d):
    lo = lax.bitcast_convert_type(lo2d.reshape(-1), jnp.uint32)
    hi = lax.bitcast_convert_type(hi2d.reshape(-1), jnp.uint32)
    u = (lax.convert_element_type(hi, jnp.uint64) << jnp.uint64(32)) | \
        lax.convert_element_type(lo, jnp.uint64)
    return lax.bitcast_convert_type(u, jnp.int64)


def _pad_rows(x2d, pad):
    z = jnp.zeros((pad, 128), jnp.int32)
    return jnp.concatenate([z, x2d, z])


def _tc_ids(p32, queue_item_ids, item_ids):
    qlo, qhi = _split_planes(queue_item_ids, PR)
    slo, shi = _split_planes(item_ids, SR)
    slo, shi = _pad_rows(slo, PADR), _pad_rows(shi, PADR)
    ids2d = jax.ShapeDtypeStruct((PR, 128), jnp.int32)
    s_lo, s_hi, n_lo, n_hi = pl.pallas_call(
        _ids_body,
        grid=(KI,),
        in_specs=[
            pl.BlockSpec((1,), lambda i: (jnp.int32(0),),
                         memory_space=pltpu.SMEM),
            pl.BlockSpec((IR, 128), _im_i0),
            pl.BlockSpec((IR, 128), _im_i0),
            pl.BlockSpec((SROWS, 128), _im_00),
            pl.BlockSpec((SROWS, 128), _im_00),
        ],
        out_specs=[
            pl.BlockSpec((IR, 128), _im_i0),
            pl.BlockSpec((IR, 128), _im_i0),
            pl.BlockSpec((IR, 128), _im_i0),
            pl.BlockSpec((IR, 128), _im_i0),
        ],
        out_shape=[ids2d, ids2d, ids2d, ids2d],
        compiler_params=pltpu.CompilerParams(dimension_semantics=("arbitrary",)),
    )(p32, qlo, qhi, slo, shi)
    return _join_planes(s_lo, s_hi), _join_planes(n_lo, n_hi)


def _fix_im(i, p_ref):
    bk = (p_ref[0] // S + lax.convert_element_type(i, jnp.int32)) % KF
    return (bk, jnp.int32(0))


def _fix_body(p_ref, nb_ref, epad_ref, out_ref):
    i = pl.program_id(0)
    p = p_ref[0]
    bk = (p // S + i) % KF
    d = bk * S - p
    s0 = jnp.where(d < 0, d + C, d)
    a1 = S + jnp.minimum(s0, B)
    a2 = jnp.maximum(S + s0 - C, 0)
    e1 = epad_ref[pl.ds(a1, S), :]
    e2 = epad_ref[pl.ds(a2, S), :]
    r = lax.broadcasted_iota(jnp.int32, (S, 1), 0)
    pos = s0 + r
    wrap = pos >= C
    posm = jnp.where(wrap, pos - C, pos)
    mask = posm < B
    val = jnp.where(wrap, e2, e1)
    out_ref[...] = jnp.where(mask, val, nb_ref[...])


def _tc_fix(p32, new_base, embeddings):
    epad = jnp.concatenate([
        jnp.zeros((S, D), jnp.float32),
        embeddings,
        jnp.zeros((S, D), jnp.float32)])
    grid_spec = pltpu.PrefetchScalarGridSpec(
        num_scalar_prefetch=1,
        grid=(NB,),
        in_specs=[
            pl.BlockSpec((S, D), _fix_im),
            pl.BlockSpec((B + 2 * S, D), lambda i, p_ref: (jnp.int32(0),
                                                           jnp.int32(0))),
        ],
        out_specs=[
            pl.BlockSpec((S, D), _fix_im),
        ],
    )
    (ne,) = pl.pallas_call(
        _fix_body,
        grid_spec=grid_spec,
        out_shape=[jax.ShapeDtypeStruct((C, D), jnp.float32)],
        input_output_aliases={1: 0},
        compiler_params=pltpu.CompilerParams(dimension_semantics=("arbitrary",)),
    )(p32, new_base, epad)
    return ne


def kernel(embeddings, item_ids, queue_embeddings, queue_item_ids, ptr):
    p32 = jnp.mod(ptr, C).astype(jnp.int32).reshape((1,))
    se, nb = _sc_copies(queue_embeddings)
    si, ni = _tc_ids(p32, queue_item_ids, item_ids)
    ne = _tc_fix(p32, nb, embeddings)
    return (se, si, ne, ni)


# speedup vs baseline: 1.2521x; 1.1551x over previous
import jax
import jax.numpy as jnp
from jax import lax
from jax.experimental import pallas as pl
from jax.experimental.pallas import tpu as pltpu
from jax.experimental.pallas import tpu_sc as plsc

C = 65536
B = 4096
D = 64
PR = C // 128
KI = 16
IR = PR // KI
SR = B // 128
PADR = 48
SROWS = SR + 2 * PADR
NT = 32
RPT = C // NT
CHUNK = 256
NCH = RPT // CHUNK
S = 512
KF = C // S
NB = B // S + 1


def _im_i0(i):
    z = jnp.int32(0)
    return (lax.convert_element_type(i, jnp.int32), z)


def _im_00(i):
    z = jnp.int32(0)
    return (z, z)


def _sc_body(q_hbm, s_hbm, n_hbm, buf0, buf1, si0, si1, so0, so1, to0, to1):
    cid = lax.axis_index("c")
    sid = lax.axis_index("s")
    wid = sid * 2 + cid
    base = wid * RPT
    bufs = (buf0, buf1)
    sin = (si0, si1)
    sout = (so0, so1)
    tout = (to0, to1)
    h_s = [None, None]
    h_n = [None, None]
    for k in range(NCH):
        b = k % 2
        if h_s[b] is not None:
            h_s[b].wait()
            h_n[b].wait()
        r0 = base + k * CHUNK
        pltpu.async_copy(q_hbm.at[pl.ds(r0, CHUNK), :], bufs[b], sin[b]).wait()
        h_s[b] = pltpu.async_copy(bufs[b], s_hbm.at[pl.ds(r0, CHUNK), :],
                                  sout[b])
        h_n[b] = pltpu.async_copy(bufs[b], n_hbm.at[pl.ds(r0, CHUNK), :],
                                  tout[b])
    for b in range(2):
        h_s[b].wait()
        h_n[b].wait()


def _sc_copies(queue_embeddings):
    emb2d = jax.ShapeDtypeStruct((C, D), jnp.float32)
    fn = pl.kernel(
        _sc_body,
        out_type=[emb2d, emb2d],
        mesh=plsc.VectorSubcoreMesh(
            core_axis_name="c", subcore_axis_name="s",
            num_cores=2, num_subcores=16),
        scratch_types=[
            pltpu.VMEM((CHUNK, D), jnp.float32),
            pltpu.VMEM((CHUNK, D), jnp.float32),
            pltpu.SemaphoreType.DMA,
            pltpu.SemaphoreType.DMA,
            pltpu.SemaphoreType.DMA,
            pltpu.SemaphoreType.DMA,
            pltpu.SemaphoreType.DMA,
            pltpu.SemaphoreType.DMA,
        ],
        compiler_params=pltpu.CompilerParams(use_tc_tiling_on_sc=True),
    )
    se, nb = fn(queue_embeddings)
    return se, nb


def _ids_body(p_ref, qlo_ref, qhi_ref, slo_ref, shi_ref,
              slo_out, shi_out, nlo_out, nhi_out):
    i = pl.program_id(0)
    p = p_ref[0]
    qlo = qlo_ref[...]
    qhi = qhi_ref[...]
    slo_out[...] = qlo
    shi_out[...] = qhi
    q = p // 128
    lam = p - q * 128
    rowg = lax.broadcasted_iota(jnp.int32, (IR, 128), 0) + i * IR
    lane = lax.broadcasted_iota(jnp.int32, (IR, 128), 1)
    g = rowg * 128 + lane
    j = g - p
    wrp = j < 0
    jm = jnp.where(wrp, j + C, j)
    mask_i = jm < B
    start_a = jnp.clip(PADR + i * IR - q - 1, 0, SROWS - (IR + 16))
    start_w = jnp.clip(PADR + i * IR - q + PR - 1, 0, SROWS - (IR + 16))
    hi_lane = lane >= lam

    def pick(src_ref):
        s_a = pltpu.roll(src_ref[pl.ds(start_a, IR + 16), :], lam, axis=1)
        s_w = pltpu.roll(src_ref[pl.ds(start_w, IR + 16), :], lam, axis=1)
        v_a = jnp.where(hi_lane, s_a[1:1 + IR], s_a[0:IR])
        v_w = jnp.where(hi_lane, s_w[1:1 + IR], s_w[0:IR])
        return jnp.where(wrp, v_w, v_a)

    nlo_out[...] = jnp.where(mask_i, pick(slo_ref), qlo)
    nhi_out[...] = jnp.where(mask_i, pick(shi_ref), qhi)


def _split_planes(x64, rows):
    u = lax.bitcast_convert_type(x64, jnp.uint64)
    lo = lax.convert_element_type(u & jnp.uint64(0xFFFFFFFF), jnp.uint32)
    hi = lax.convert_element_type(u >> jnp.uint64(32), jnp.uint32)
    lo = lax.bitcast_convert_type(lo, jnp.int32).reshape(rows, 128)
    hi = lax.bitcast_convert_type(hi, jnp.int32).reshape(rows, 128)
    return lo, hi


def _join_planes(lo2d, hi2d):
    lo = lax.bitcast_convert_type(lo2d.reshape(-1), jnp.uint32)
    hi = lax.bitcast_convert_type(hi2d.reshape(-1), jnp.uint32)
    u = (lax.convert_element_type(hi, jnp.uint64) << jnp.uint64(32)) | \
        lax.convert_element_type(lo, jnp.uint64)
    return lax.bitcast_convert_type(u, jnp.int64)


def _pad_rows(x2d, pad):
    z = jnp.zeros((pad, 128), jnp.int32)
    return jnp.concatenate([z, x2d, z])


def _tc_ids(p32, queue_item_ids, item_ids):
    qlo, qhi = _split_planes(queue_item_ids, PR)
    slo, shi = _split_planes(item_ids, SR)
    slo, shi = _pad_rows(slo, PADR), _pad_rows(shi, PADR)
    ids2d = jax.ShapeDtypeStruct((PR, 128), jnp.int32)
    s_lo, s_hi, n_lo, n_hi = pl.pallas_call(
        _ids_body,
        grid=(KI,),
        in_specs=[
            pl.BlockSpec((1,), lambda i: (jnp.int32(0),),
                         memory_space=pltpu.SMEM),
            pl.BlockSpec((IR, 128), _im_i0),
            pl.BlockSpec((IR, 128), _im_i0),
            pl.BlockSpec((SROWS, 128), _im_00),
            pl.BlockSpec((SROWS, 128), _im_00),
        ],
        out_specs=[
            pl.BlockSpec((IR, 128), _im_i0),
            pl.BlockSpec((IR, 128), _im_i0),
            pl.BlockSpec((IR, 128), _im_i0),
            pl.BlockSpec((IR, 128), _im_i0),
        ],
        out_shape=[ids2d, ids2d, ids2d, ids2d],
        compiler_params=pltpu.CompilerParams(dimension_semantics=("arbitrary",)),
    )(p32, qlo, qhi, slo, shi)
    return _join_planes(s_lo, s_hi), _join_planes(n_lo, n_hi)


def _fix_im(i, p_ref):
    bk = (p_ref[0] // S + lax.convert_element_type(i, jnp.int32)) % KF
    return (bk, jnp.int32(0))


def _fix_body(p_ref, nb_ref, epad_ref, out_ref):
    i = pl.program_id(0)
    p = p_ref[0]
    bk = (p // S + i) % KF
    d = bk * S - p
    s0 = jnp.where(d < 0, d + C, d)
    a1 = S + jnp.minimum(s0, B)
    a2 = jnp.maximum(S + s0 - C, 0)
    e1 = epad_ref[pl.ds(a1, S), :]
    e2 = epad_ref[pl.ds(a2, S), :]
    r = lax.broadcasted_iota(jnp.int32, (S, 1), 0)
    pos = s0 + r
    wrap = pos >= C
    posm = jnp.where(wrap, pos - C, pos)
    mask = posm < B
    val = jnp.where(wrap, e2, e1)
    out_ref[...] = jnp.where(mask, val, nb_ref[...])


def _tc_fix(p32, new_base, embeddings):
    epad = jnp.concatenate([
        jnp.zeros((S, D), jnp.float32),
        embeddings,
        jnp.zeros((S, D), jnp.float32)])
    grid_spec = pltpu.PrefetchScalarGridSpec(
        num_scalar_prefetch=1,
        grid=(NB,),
        in_specs=[
            pl.BlockSpec((S, D), _fix_im),
            pl.BlockSpec((B + 2 * S, D), lambda i, p_ref: (jnp.int32(0),
                                                           jnp.int32(0))),
        ],
        out_specs=[
            pl.BlockSpec((S, D), _fix_im),
        ],
    )
    (ne,) = pl.pallas_call(
        _fix_body,
        grid_spec=grid_spec,
        out_shape=[jax.ShapeDtypeStruct((C, D), jnp.float32)],
        input_output_aliases={1: 0},
        compiler_params=pltpu.CompilerParams(dimension_semantics=("arbitrary",)),
    )(p32, new_base, epad)
    return ne


def kernel(embeddings, item_ids, queue_embeddings, queue_item_ids, ptr):
    p32 = jnp.mod(ptr, C).astype(jnp.int32).reshape((1,))
    se, nb = _sc_copies(queue_embeddings)
    si, ni = _tc_ids(p32, queue_item_ids, item_ids)
    ne = _tc_fix(p32, nb, embeddings)
    return (se, si, ne, ni)
